# Initial kernel scaffold; baseline (speedup 1.0000x reference)
import jax
import jax.numpy as jnp
from jax.experimental import pallas as pl


def kernel(loc, depot, W_init, b_init, W_nbr, b_nbr, W_fin, b_fin, W_dep, b_dep, bn_w, bn_b):
    raise NotImplementedError("write your pallas kernel here")


# speedup vs baseline: 6.6383x; 6.6383x over previous
import jax
import jax.numpy as jnp
from jax.experimental import pallas as pl

_B, _N, _E = 16, 1000, 128
_RC = 200
_NC = _N // _RC
_K = 10


def _knn_fe_body(xq_ref, xbT_ref, x0T_ref, wc_ref, fe_ref, stats_ref):
    b = pl.program_id(0)
    c = pl.program_id(1)
    q0 = xq_ref[0, :, 0:1]
    q1 = xq_ref[0, :, 1:2]
    k0 = xbT_ref[0, 0:1, :]
    k1 = xbT_ref[0, 1:2, :]
    d0 = q0 - k0
    d1 = q1 - k1
    dist2 = d0 * d0 + d1 * d1
    iota = jax.lax.broadcasted_iota(jnp.int32, (_RC, _N), 1)
    big = jnp.float32(jnp.inf)

    def body(_, carry):
        work, acc = carry
        m = jnp.min(work, axis=1, keepdims=True)
        cand = jnp.where(work == m, iota, _N)
        idx = jnp.min(cand, axis=1, keepdims=True)
        onehot = iota == idx
        acc = acc + jnp.where(onehot, 1.0, 0.0)
        work = jnp.where(onehot, big, work)
        return work, acc

    _, acc = jax.lax.fori_loop(
        0, _K, body, (dist2, jnp.zeros((_RC, _N), jnp.float32)))

    g0 = x0T_ref[0, 0:1, :]
    g1 = x0T_ref[0, 1:2, :]
    s0 = jnp.sum(acc * g0, axis=1, keepdims=True)
    s1 = jnp.sum(acc * g1, axis=1, keepdims=True)
    t0 = s0 - jnp.float32(_K) * q0
    t1 = s1 - jnp.float32(_K) * q1
    fe = (q0 * wc_ref[0:1, :] + q1 * wc_ref[1:2, :]
          + t0 * wc_ref[2:3, :] + t1 * wc_ref[3:4, :])
    fe_ref[0, :, :] = fe

    @pl.when((b == 0) & (c == 0))
    def _():
        stats_ref[:, :] = jnp.zeros((8, _E), jnp.float32)

    stats_ref[0:1, :] += jnp.sum(fe, axis=0, keepdims=True)
    stats_ref[1:2, :] += jnp.sum(fe * fe, axis=0, keepdims=True)


def _bn_body(fe_ref, stats_ref, dep_ref, wdep_ref, bdep_ref, bnw_ref, bnb_ref,
             hb_ref, hd_ref, mh_ref):
    inv_n = jnp.float32(1.0 / (_B * _N))
    mean = stats_ref[0:1, :] * inv_n
    ex2 = stats_ref[1:2, :] * inv_n
    var = ex2 - mean * mean
    scale = jax.lax.rsqrt(var + jnp.float32(1e-5)) * bnw_ref[0:1, :]
    fe = fe_ref[0]
    normed = (fe - mean) * scale + bnb_ref[0:1, :]
    hb = jnp.where(normed >= 0, normed, jnp.float32(0.01) * normed)
    hb_ref[0] = hb
    dd0 = dep_ref[0, :, 0:1]
    dd1 = dep_ref[0, :, 1:2]
    dep = dd0 * wdep_ref[0:1, :] + dd1 * wdep_ref[1:2, :] + bdep_ref[0:1, :]
    hd = jnp.where(dep >= 0, dep, jnp.float32(0.01) * dep)
    hd_ref[0] = hd
    mh_ref[0] = (jnp.sum(hb, axis=0, keepdims=True) + hd) / jnp.float32(_N + 1)


def kernel(loc, depot, W_init, b_init, W_nbr, b_nbr, W_fin, b_fin,
           W_dep, b_dep, bn_w, bn_b):
    locT = jnp.transpose(loc, (0, 2, 1))
    wc = jnp.concatenate([W_init @ W_fin, W_nbr @ W_fin], axis=0)

    fe, stats = pl.pallas_call(
        _knn_fe_body,
        grid=(_B, _NC),
        in_specs=[
            pl.BlockSpec((1, _RC, 2), lambda b, c: (b, c, 0)),
            pl.BlockSpec((1, 2, _N), lambda b, c: (b, 0, 0)),
            pl.BlockSpec((1, 2, _N), lambda b, c: (0, 0, 0)),
            pl.BlockSpec((4, _E), lambda b, c: (0, 0)),
        ],
        out_specs=[
            pl.BlockSpec((1, _RC, _E), lambda b, c: (b, c, 0)),
            pl.BlockSpec((8, _E), lambda b, c: (0, 0)),
        ],
        out_shape=[
            jax.ShapeDtypeStruct((_B, _N, _E), jnp.float32),
            jax.ShapeDtypeStruct((8, _E), jnp.float32),
        ],
    )(loc, locT, locT, wc)

    hb, hd, mh = pl.pallas_call(
        _bn_body,
        grid=(_B,),
        in_specs=[
            pl.BlockSpec((1, _N, _E), lambda b: (b, 0, 0)),
            pl.BlockSpec((8, _E), lambda b: (0, 0)),
            pl.BlockSpec((1, 1, 2), lambda b: (b, 0, 0)),
            pl.BlockSpec((2, _E), lambda b: (0, 0)),
            pl.BlockSpec((1, _E), lambda b: (0, 0)),
            pl.BlockSpec((1, _E), lambda b: (0, 0)),
            pl.BlockSpec((1, _E), lambda b: (0, 0)),
        ],
        out_specs=[
            pl.BlockSpec((1, _N, _E), lambda b: (b, 0, 0)),
            pl.BlockSpec((1, 1, _E), lambda b: (b, 0, 0)),
            pl.BlockSpec((1, 1, _E), lambda b: (b, 0, 0)),
        ],
        out_shape=[
            jax.ShapeDtypeStruct((_B, _N, _E), jnp.float32),
            jax.ShapeDtypeStruct((_B, 1, _E), jnp.float32),
            jax.ShapeDtypeStruct((_B, 1, _E), jnp.float32),
        ],
    )(fe, stats, depot, W_dep, b_dep[None, :], bn_w[None, :], bn_b[None, :])

    h = jnp.concatenate([hd, hb], axis=1)
    return h, mh[:, 0, :]


# speedup vs baseline: 6.6420x; 1.0005x over previous
import jax
import jax.numpy as jnp
from jax.experimental import pallas as pl

_B, _N, _E = 16, 1000, 128
_RC = 200
_NC = _N // _RC
_K = 10


def _knn_fe_body(xq_ref, xbT_ref, x0T_ref, wc_ref, fe_ref, stats_ref):
    b = pl.program_id(0)
    c = pl.program_id(1)
    q0 = xq_ref[0, :, 0:1]
    q1 = xq_ref[0, :, 1:2]
    k0 = xbT_ref[0, 0:1, :]
    k1 = xbT_ref[0, 1:2, :]
    d0 = q0 - k0
    d1 = q1 - k1
    dist2 = d0 * d0 + d1 * d1
    iota = jax.lax.broadcasted_iota(jnp.int32, (_RC, _N), 1)
    big = jnp.float32(jnp.inf)
    g0 = x0T_ref[0, 0:1, :]
    g1 = x0T_ref[0, 1:2, :]
    zero = jnp.float32(0.0)

    def extract(work, s0, s1, m):
        cand = jnp.where(work == m, iota, _N)
        idx = jnp.min(cand, axis=1, keepdims=True)
        onehot = cand == idx
        s0 = s0 + jnp.sum(jnp.where(onehot, g0, zero), axis=1, keepdims=True)
        s1 = s1 + jnp.sum(jnp.where(onehot, g1, zero), axis=1, keepdims=True)
        work = jnp.where(onehot, big, work)
        return work, s0, s1

    work, s0, s1 = extract(dist2,
                           jnp.zeros((_RC, 1), jnp.float32),
                           jnp.zeros((_RC, 1), jnp.float32),
                           zero)

    def body(_, carry):
        work, s0, s1 = carry
        m = jnp.min(work, axis=1, keepdims=True)
        return extract(work, s0, s1, m)

    _, s0, s1 = jax.lax.fori_loop(0, _K - 1, body, (work, s0, s1))
    t0 = s0 - jnp.float32(_K) * q0
    t1 = s1 - jnp.float32(_K) * q1
    fe = (q0 * wc_ref[0:1, :] + q1 * wc_ref[1:2, :]
          + t0 * wc_ref[2:3, :] + t1 * wc_ref[3:4, :])
    fe_ref[0, :, :] = fe

    @pl.when((b == 0) & (c == 0))
    def _():
        stats_ref[:, :] = jnp.zeros((8, _E), jnp.float32)

    stats_ref[0:1, :] += jnp.sum(fe, axis=0, keepdims=True)
    stats_ref[1:2, :] += jnp.sum(fe * fe, axis=0, keepdims=True)


def _bn_body(fe_ref, stats_ref, dep_ref, wdep_ref, bdep_ref, bnw_ref, bnb_ref,
             hb_ref, hd_ref, mh_ref):
    inv_n = jnp.float32(1.0 / (_B * _N))
    mean = stats_ref[0:1, :] * inv_n
    ex2 = stats_ref[1:2, :] * inv_n
    var = ex2 - mean * mean
    scale = jax.lax.rsqrt(var + jnp.float32(1e-5)) * bnw_ref[0:1, :]
    fe = fe_ref[0]
    normed = (fe - mean) * scale + bnb_ref[0:1, :]
    hb = jnp.where(normed >= 0, normed, jnp.float32(0.01) * normed)
    hb_ref[0] = hb
    dd0 = dep_ref[0, :, 0:1]
    dd1 = dep_ref[0, :, 1:2]
    dep = dd0 * wdep_ref[0:1, :] + dd1 * wdep_ref[1:2, :] + bdep_ref[0:1, :]
    hd = jnp.where(dep >= 0, dep, jnp.float32(0.01) * dep)
    hd_ref[0] = hd
    mh_ref[0] = (jnp.sum(hb, axis=0, keepdims=True) + hd) / jnp.float32(_N + 1)


def kernel(loc, depot, W_init, b_init, W_nbr, b_nbr, W_fin, b_fin,
           W_dep, b_dep, bn_w, bn_b):
    locT = jnp.transpose(loc, (0, 2, 1))
    wc = jnp.concatenate([W_init @ W_fin, W_nbr @ W_fin], axis=0)

    fe, stats = pl.pallas_call(
        _knn_fe_body,
        grid=(_B, _NC),
        in_specs=[
            pl.BlockSpec((1, _RC, 2), lambda b, c: (b, c, 0)),
            pl.BlockSpec((1, 2, _N), lambda b, c: (b, 0, 0)),
            pl.BlockSpec((1, 2, _N), lambda b, c: (0, 0, 0)),
            pl.BlockSpec((4, _E), lambda b, c: (0, 0)),
        ],
        out_specs=[
            pl.BlockSpec((1, _RC, _E), lambda b, c: (b, c, 0)),
            pl.BlockSpec((8, _E), lambda b, c: (0, 0)),
        ],
        out_shape=[
            jax.ShapeDtypeStruct((_B, _N, _E), jnp.float32),
            jax.ShapeDtypeStruct((8, _E), jnp.float32),
        ],
    )(loc, locT, locT, wc)

    hb, hd, mh = pl.pallas_call(
        _bn_body,
        grid=(_B,),
        in_specs=[
            pl.BlockSpec((1, _N, _E), lambda b: (b, 0, 0)),
            pl.BlockSpec((8, _E), lambda b: (0, 0)),
            pl.BlockSpec((1, 1, 2), lambda b: (b, 0, 0)),
            pl.BlockSpec((2, _E), lambda b: (0, 0)),
            pl.BlockSpec((1, _E), lambda b: (0, 0)),
            pl.BlockSpec((1, _E), lambda b: (0, 0)),
            pl.BlockSpec((1, _E), lambda b: (0, 0)),
        ],
        out_specs=[
            pl.BlockSpec((1, _N, _E), lambda b: (b, 0, 0)),
            pl.BlockSpec((1, 1, _E), lambda b: (b, 0, 0)),
            pl.BlockSpec((1, 1, _E), lambda b: (b, 0, 0)),
        ],
        out_shape=[
            jax.ShapeDtypeStruct((_B, _N, _E), jnp.float32),
            jax.ShapeDtypeStruct((_B, 1, _E), jnp.float32),
            jax.ShapeDtypeStruct((_B, 1, _E), jnp.float32),
        ],
    )(fe, stats, depot, W_dep, b_dep[None, :], bn_w[None, :], bn_b[None, :])

    h = jnp.concatenate([hd, hb], axis=1)
    return h, mh[:, 0, :]


# speedup vs baseline: 14.2883x; 2.1512x over previous
import jax
import jax.numpy as jnp
from jax.experimental import pallas as pl

_B, _N, _E = 16, 1000, 128
_RC = 200
_NC = _N // _RC
_K = 10


def _knn_fe_body(xq_ref, xbT_ref, x0_ref, wc_ref, fe_ref, stats_ref):
    b = pl.program_id(0)
    c = pl.program_id(1)
    q0 = xq_ref[0, :, 0:1]
    q1 = xq_ref[0, :, 1:2]
    k0 = xbT_ref[0, 0:1, :]
    k1 = xbT_ref[0, 1:2, :]
    d0 = q0 - k0
    d1 = q1 - k1
    dist2 = d0 * d0 + d1 * d1
    iota = jax.lax.broadcasted_iota(jnp.int32, (_RC, _N), 1).astype(jnp.float32)
    big = jnp.float32(jnp.inf)
    bigi = jnp.float32(2e9)
    zero = jnp.float32(0.0)
    one = jnp.float32(1.0)

    def extract(work, acc, m):
        cand = jnp.where(work == m, iota, bigi)
        idx = jnp.min(cand, axis=1, keepdims=True)
        onehot = cand == idx
        acc = acc + jnp.where(onehot, one, zero)
        work = jnp.where(onehot, big, work)
        return work, acc

    work, acc = extract(dist2, jnp.zeros((_RC, _N), jnp.float32), zero)
    for _ in range(_K - 1):
        m = jnp.min(work, axis=1, keepdims=True)
        work, acc = extract(work, acc, m)

    s = jnp.dot(acc, x0_ref[0], preferred_element_type=jnp.float32)
    t0 = s[:, 0:1] - jnp.float32(_K) * q0
    t1 = s[:, 1:2] - jnp.float32(_K) * q1
    fe = (q0 * wc_ref[0:1, :] + q1 * wc_ref[1:2, :]
          + t0 * wc_ref[2:3, :] + t1 * wc_ref[3:4, :])
    fe_ref[0, :, :] = fe

    @pl.when((b == 0) & (c == 0))
    def _():
        stats_ref[:, :] = jnp.zeros((8, _E), jnp.float32)

    stats_ref[0:1, :] += jnp.sum(fe, axis=0, keepdims=True)
    stats_ref[1:2, :] += jnp.sum(fe * fe, axis=0, keepdims=True)


def _bn_body(fe_ref, stats_ref, dep_ref, wdep_ref, bdep_ref, bnw_ref, bnb_ref,
             hb_ref, hd_ref, mh_ref):
    inv_n = jnp.float32(1.0 / (_B * _N))
    mean = stats_ref[0:1, :] * inv_n
    ex2 = stats_ref[1:2, :] * inv_n
    var = ex2 - mean * mean
    scale = jax.lax.rsqrt(var + jnp.float32(1e-5)) * bnw_ref[0:1, :]
    fe = fe_ref[0]
    normed = (fe - mean) * scale + bnb_ref[0:1, :]
    hb = jnp.where(normed >= 0, normed, jnp.float32(0.01) * normed)
    hb_ref[0] = hb
    dd0 = dep_ref[0, :, 0:1]
    dd1 = dep_ref[0, :, 1:2]
    dep = dd0 * wdep_ref[0:1, :] + dd1 * wdep_ref[1:2, :] + bdep_ref[0:1, :]
    hd = jnp.where(dep >= 0, dep, jnp.float32(0.01) * dep)
    hd_ref[0] = hd
    mh_ref[0] = (jnp.sum(hb, axis=0, keepdims=True) + hd) / jnp.float32(_N + 1)


def kernel(loc, depot, W_init, b_init, W_nbr, b_nbr, W_fin, b_fin,
           W_dep, b_dep, bn_w, bn_b):
    locT = jnp.transpose(loc, (0, 2, 1))
    wc = jnp.concatenate([W_init @ W_fin, W_nbr @ W_fin], axis=0)

    fe, stats = pl.pallas_call(
        _knn_fe_body,
        grid=(_B, _NC),
        in_specs=[
            pl.BlockSpec((1, _RC, 2), lambda b, c: (b, c, 0)),
            pl.BlockSpec((1, 2, _N), lambda b, c: (b, 0, 0)),
            pl.BlockSpec((1, _N, 2), lambda b, c: (0, 0, 0)),
            pl.BlockSpec((4, _E), lambda b, c: (0, 0)),
        ],
        out_specs=[
            pl.BlockSpec((1, _RC, _E), lambda b, c: (b, c, 0)),
            pl.BlockSpec((8, _E), lambda b, c: (0, 0)),
        ],
        out_shape=[
            jax.ShapeDtypeStruct((_B, _N, _E), jnp.float32),
            jax.ShapeDtypeStruct((8, _E), jnp.float32),
        ],
    )(loc, locT, loc, wc)

    hb, hd, mh = pl.pallas_call(
        _bn_body,
        grid=(_B,),
        in_specs=[
            pl.BlockSpec((1, _N, _E), lambda b: (b, 0, 0)),
            pl.BlockSpec((8, _E), lambda b: (0, 0)),
            pl.BlockSpec((1, 1, 2), lambda b: (b, 0, 0)),
            pl.BlockSpec((2, _E), lambda b: (0, 0)),
            pl.BlockSpec((1, _E), lambda b: (0, 0)),
            pl.BlockSpec((1, _E), lambda b: (0, 0)),
            pl.BlockSpec((1, _E), lambda b: (0, 0)),
        ],
        out_specs=[
            pl.BlockSpec((1, _N, _E), lambda b: (b, 0, 0)),
            pl.BlockSpec((1, 1, _E), lambda b: (b, 0, 0)),
            pl.BlockSpec((1, 1, _E), lambda b: (b, 0, 0)),
        ],
        out_shape=[
            jax.ShapeDtypeStruct((_B, _N, _E), jnp.float32),
            jax.ShapeDtypeStruct((_B, 1, _E), jnp.float32),
            jax.ShapeDtypeStruct((_B, 1, _E), jnp.float32),
        ],
    )(fe, stats, depot, W_dep, b_dep[None, :], bn_w[None, :], bn_b[None, :])

    h = jnp.concatenate([hd, hb], axis=1)
    return h, mh[:, 0, :]


# speedup vs baseline: 16.7364x; 1.1713x over previous
import jax
import jax.numpy as jnp
from jax.experimental import pallas as pl

_B, _N, _E = 16, 1000, 128
_RC = 200
_NC = _N // _RC
_K = 10


def _knn_fe_body(xq_ref, xbT_ref, x0_ref, wc_ref, fe_ref, stats_ref):
    b = pl.program_id(0)
    c = pl.program_id(1)
    q0 = xq_ref[0, :, 0:1]
    q1 = xq_ref[0, :, 1:2]
    k0 = xbT_ref[0, 0:1, :]
    k1 = xbT_ref[0, 1:2, :]
    d0 = q0 - k0
    d1 = q1 - k1
    dist2 = d0 * d0 + d1 * d1
    iota = jax.lax.broadcasted_iota(jnp.int32, (_RC, _N), 1).astype(jnp.float32)
    big = jnp.float32(jnp.inf)
    bigi = jnp.float32(2e9)
    zero = jnp.float32(0.0)
    one = jnp.float32(1.0)

    def extract(work, m):
        cand = jnp.where(work == m, iota, bigi)
        idx = jnp.min(cand, axis=1, keepdims=True)
        return jnp.where(cand == idx, big, work)

    work = extract(dist2, zero)
    for _ in range(_K - 1):
        m = jnp.min(work, axis=1, keepdims=True)
        work = extract(work, m)

    acc = jnp.where(work == big, one, zero)
    s = jnp.dot(acc, x0_ref[0], preferred_element_type=jnp.float32)
    t0 = s[:, 0:1] - jnp.float32(_K) * q0
    t1 = s[:, 1:2] - jnp.float32(_K) * q1
    fe = (q0 * wc_ref[0:1, :] + q1 * wc_ref[1:2, :]
          + t0 * wc_ref[2:3, :] + t1 * wc_ref[3:4, :])
    fe_ref[0, :, :] = fe

    @pl.when((b == 0) & (c == 0))
    def _():
        stats_ref[:, :] = jnp.zeros((8, _E), jnp.float32)

    stats_ref[0:1, :] += jnp.sum(fe, axis=0, keepdims=True)
    stats_ref[1:2, :] += jnp.sum(fe * fe, axis=0, keepdims=True)


def _bn_body(fe_ref, stats_ref, dep_ref, wdep_ref, bdep_ref, bnw_ref, bnb_ref,
             hb_ref, hd_ref, mh_ref):
    inv_n = jnp.float32(1.0 / (_B * _N))
    mean = stats_ref[0:1, :] * inv_n
    ex2 = stats_ref[1:2, :] * inv_n
    var = ex2 - mean * mean
    scale = jax.lax.rsqrt(var + jnp.float32(1e-5)) * bnw_ref[0:1, :]
    fe = fe_ref[0]
    normed = (fe - mean) * scale + bnb_ref[0:1, :]
    hb = jnp.where(normed >= 0, normed, jnp.float32(0.01) * normed)
    hb_ref[0] = hb
    dd0 = dep_ref[0, :, 0:1]
    dd1 = dep_ref[0, :, 1:2]
    dep = dd0 * wdep_ref[0:1, :] + dd1 * wdep_ref[1:2, :] + bdep_ref[0:1, :]
    hd = jnp.where(dep >= 0, dep, jnp.float32(0.01) * dep)
    hd_ref[0] = hd
    mh_ref[0] = (jnp.sum(hb, axis=0, keepdims=True) + hd) / jnp.float32(_N + 1)


def kernel(loc, depot, W_init, b_init, W_nbr, b_nbr, W_fin, b_fin,
           W_dep, b_dep, bn_w, bn_b):
    locT = jnp.transpose(loc, (0, 2, 1))
    wc = jnp.concatenate([W_init @ W_fin, W_nbr @ W_fin], axis=0)

    fe, stats = pl.pallas_call(
        _knn_fe_body,
        grid=(_B, _NC),
        in_specs=[
            pl.BlockSpec((1, _RC, 2), lambda b, c: (b, c, 0)),
            pl.BlockSpec((1, 2, _N), lambda b, c: (b, 0, 0)),
            pl.BlockSpec((1, _N, 2), lambda b, c: (0, 0, 0)),
            pl.BlockSpec((4, _E), lambda b, c: (0, 0)),
        ],
        out_specs=[
            pl.BlockSpec((1, _RC, _E), lambda b, c: (b, c, 0)),
            pl.BlockSpec((8, _E), lambda b, c: (0, 0)),
        ],
        out_shape=[
            jax.ShapeDtypeStruct((_B, _N, _E), jnp.float32),
            jax.ShapeDtypeStruct((8, _E), jnp.float32),
        ],
    )(loc, locT, loc, wc)

    hb, hd, mh = pl.pallas_call(
        _bn_body,
        grid=(_B,),
        in_specs=[
            pl.BlockSpec((1, _N, _E), lambda b: (b, 0, 0)),
            pl.BlockSpec((8, _E), lambda b: (0, 0)),
            pl.BlockSpec((1, 1, 2), lambda b: (b, 0, 0)),
            pl.BlockSpec((2, _E), lambda b: (0, 0)),
            pl.BlockSpec((1, _E), lambda b: (0, 0)),
            pl.BlockSpec((1, _E), lambda b: (0, 0)),
            pl.BlockSpec((1, _E), lambda b: (0, 0)),
        ],
        out_specs=[
            pl.BlockSpec((1, _N, _E), lambda b: (b, 0, 0)),
            pl.BlockSpec((1, 1, _E), lambda b: (b, 0, 0)),
            pl.BlockSpec((1, 1, _E), lambda b: (b, 0, 0)),
        ],
        out_shape=[
            jax.ShapeDtypeStruct((_B, _N, _E), jnp.float32),
            jax.ShapeDtypeStruct((_B, 1, _E), jnp.float32),
            jax.ShapeDtypeStruct((_B, 1, _E), jnp.float32),
        ],
    )(fe, stats, depot, W_dep, b_dep[None, :], bn_w[None, :], bn_b[None, :])

    h = jnp.concatenate([hd, hb], axis=1)
    return h, mh[:, 0, :]
